# Initial kernel scaffold; baseline (speedup 1.0000x reference)
import jax
import jax.numpy as jnp
from jax.experimental import pallas as pl


def kernel(uttr_ids_list, position_ids_list, word_emb, pos_emb, ln_gamma, ln_beta):
    raise NotImplementedError("write your pallas kernel here")


# speedup vs baseline: 4.6199x; 4.6199x over previous
import functools

import numpy as np

import jax
import jax.numpy as jnp
from jax import lax
from jax.experimental import pallas as pl
from jax.experimental.pallas import tpu as pltpu
from jax.experimental.pallas import tpu_sc as plsc

HIDDEN = 128
PAD = 0
EPS = 1e-5

NC = 2
NS = 16
NW = NC * NS
L = 16
NBLK = HIDDEN // L

C = 128

_RSQRT_MAGIC = np.int32(0x5F3759DF)


_GATHER_DNUMS = lax.GatherDimensionNumbers(
    offset_dims=(), collapsed_slice_dims=(0,), start_index_map=(0,))


def _splat_last(v):
    last = jnp.full((L, 1), L - 1, jnp.int32)
    return lax.gather(v, last, _GATHER_DNUMS, (1,),
                      mode=lax.GatherScatterMode.PROMISE_IN_BOUNDS)


def _rsqrt_vec(a):
    ai = lax.bitcast_convert_type(a, jnp.int32)
    y = lax.bitcast_convert_type(_RSQRT_MAGIC - (ai >> 1), jnp.float32)
    ha = a * 0.5
    for _ in range(3):
        y = y * (1.5 - ha * y * y)
    return y


def _make_kernel(n_tokens):
    assert n_tokens % (NW * C) == 0
    per_w = n_tokens // NW
    n_chunks = per_w // C

    mesh = plsc.VectorSubcoreMesh(
        core_axis_name="c", subcore_axis_name="s",
        num_cores=NC, num_subcores=NS,
    )

    @functools.partial(
        pl.kernel,
        out_type=(
            jax.ShapeDtypeStruct((n_tokens, HIDDEN), jnp.float32),
            jax.ShapeDtypeStruct((n_tokens,), jnp.int32),
        ),
        mesh=mesh,
        compiler_params=pltpu.CompilerParams(needs_layout_passes=False),
        scratch_types=[
            pltpu.VMEM((C,), jnp.int32),
            pltpu.VMEM((C,), jnp.int32),
            pltpu.VMEM((C, HIDDEN), jnp.float32),
            pltpu.VMEM((C, HIDDEN), jnp.float32),
            pltpu.VMEM((C,), jnp.int32),
            pltpu.VMEM((HIDDEN,), jnp.float32),
            pltpu.VMEM((HIDDEN,), jnp.float32),
            pltpu.SemaphoreType.DMA,
        ],
    )
    def emb_kernel(idw_hbm, idp_hbm, wtab_hbm, ptab_hbm, g_hbm, b_hbm,
                   out_hbm, mask_hbm,
                   idw_v, idp_v, xrows, orows, mvec, gv, bv, sem):
        wid = lax.axis_index("s") * NC + lax.axis_index("c")
        base = wid * per_w

        pltpu.sync_copy(g_hbm, gv)
        pltpu.sync_copy(b_hbm, bv)
        gs = [gv[pl.ds(L * e, L)] for e in range(NBLK)]
        bs = [bv[pl.ds(L * e, L)] for e in range(NBLK)]

        def chunk_body(ci, carry):
            off = base + ci * C
            pltpu.sync_copy(idw_hbm.at[pl.ds(off, C)], idw_v)
            pltpu.sync_copy(idp_hbm.at[pl.ds(off, C)], idp_v)
            pltpu.async_copy(wtab_hbm.at[idw_v], xrows, sem).wait()
            pltpu.async_copy(ptab_hbm.at[idp_v], xrows, sem, add=True).wait()

            def mask_body(g, c2):
                v = idw_v[pl.ds(g * L, L)]
                mvec[pl.ds(g * L, L)] = 1 - jnp.minimum(v, 1)
                return c2

            lax.fori_loop(0, C // L, mask_body, 0)

            def tok_body(t, c2):
                xs = [xrows[t, pl.ds(L * e, L)] for e in range(NBLK)]
                s = xs[0]
                ssq = xs[0] * xs[0]
                for e in range(1, NBLK):
                    s = s + xs[e]
                    ssq = ssq + xs[e] * xs[e]
                mean = jnp.sum(s) * (1.0 / HIDDEN)
                var = jnp.sum(ssq) * (1.0 / HIDDEN) - mean * mean
                inv = _rsqrt_vec(var + EPS)
                for e in range(NBLK):
                    t1 = gs[e] * inv
                    orows[t, pl.ds(L * e, L)] = (xs[e] - mean) * t1 + bs[e]
                return c2

            lax.fori_loop(0, C, tok_body, 0)

            pltpu.sync_copy(orows, out_hbm.at[pl.ds(off, C)])
            pltpu.sync_copy(mvec, mask_hbm.at[pl.ds(off, C)])
            return carry

        lax.fori_loop(0, n_chunks, chunk_body, 0)

    return emb_kernel


@jax.jit
def _run(idw, idp, word_emb, pos_emb, ln_gamma, ln_beta):
    n_tokens = idw.shape[0]
    w = word_emb.at[PAD].set(0.0)
    p = pos_emb.at[PAD].set(0.0)
    return _make_kernel(n_tokens)(idw, idp, w, p, ln_gamma, ln_beta)


def kernel(uttr_ids_list, position_ids_list, word_emb, pos_emb, ln_gamma,
           ln_beta):
    B, S = uttr_ids_list.shape
    n = B * S
    out, mask = _run(uttr_ids_list.reshape(n), position_ids_list.reshape(n),
                     word_emb, pos_emb, ln_gamma, ln_beta)
    return out.reshape(B, S, HIDDEN), mask.reshape(B, S).astype(bool)
